# Initial kernel scaffold; baseline (speedup 1.0000x reference)
import jax
import jax.numpy as jnp
from jax.experimental import pallas as pl


def kernel(P_batch, Q_batch, projections):
    raise NotImplementedError("write your pallas kernel here")


# speedup vs baseline: 4.3748x; 4.3748x over previous
import jax
import jax.numpy as jnp
from jax import lax
from jax.experimental import pallas as pl
from jax.experimental.pallas import tpu as pltpu

_N = 8192
_D = 128
_L = 100
_C = 64
_NCHUNK = 2


def _swd_kernel(p_ref, q_ref, proj_ref, out_ref):
    P = p_ref[0]
    Q = q_ref[0]
    proj = proj_ref[0]
    Xp = jnp.dot(P, proj, preferred_element_type=jnp.float32)
    Yp = jnp.dot(Q, proj, preferred_element_type=jnp.float32)
    x = jnp.concatenate([Xp, Yp], axis=1)

    iota = lax.broadcasted_iota(jnp.int32, (_N, 1), 0)
    k = 2
    while k <= _N:
        asc = (iota & k) == 0
        j = k // 2
        while j >= 1:
            bit_clear = (iota & j) == 0
            up = jnp.roll(x, -j, axis=0)
            dn = jnp.roll(x, j, axis=0)
            partner = jnp.where(bit_clear, up, dn)
            lo = jnp.minimum(x, partner)
            hi = jnp.maximum(x, partner)
            x = jnp.where(asc == bit_clear, lo, hi)
            j //= 2
        k *= 2

    d = x[:, :_C] - x[:, _C:]
    m = jnp.mean(d * d, axis=0, keepdims=True)
    out_ref[0, 0] = jnp.concatenate([m, jnp.zeros((1, 2 * _C - _C), jnp.float32)], axis=1)


def kernel(P_batch, Q_batch, projections):
    B = P_batch.shape[0]
    projp = jnp.zeros((_D, _NCHUNK * _C), jnp.float32).at[:, :_L].set(projections)
    projc = projp.reshape(_D, _NCHUNK, _C).transpose(1, 0, 2)
    wpp = pl.pallas_call(
        _swd_kernel,
        grid=(B, _NCHUNK),
        in_specs=[
            pl.BlockSpec((1, _N, _D), lambda b, c: (b, 0, 0)),
            pl.BlockSpec((1, _N, _D), lambda b, c: (b, 0, 0)),
            pl.BlockSpec((1, _D, _C), lambda b, c: (c, 0, 0)),
        ],
        out_specs=pl.BlockSpec((1, 1, 1, 2 * _C), lambda b, c: (b, c, 0, 0)),
        out_shape=jax.ShapeDtypeStruct((B, _NCHUNK, 1, 2 * _C), jnp.float32),
        compiler_params=pltpu.CompilerParams(
            vmem_limit_bytes=110 * 1024 * 1024,
        ),
    )(P_batch, Q_batch, projc)
    wpp_full = jnp.concatenate([wpp[:, 0, 0, :_C], wpp[:, 1, 0, :_C]], axis=1)
    swd = jnp.sqrt(jnp.mean(wpp_full[:, :_L], axis=1))
    return jnp.sum(swd) / B
